# baseline (speedup 1.0000x reference)
import jax
import jax.numpy as jnp
from jax.experimental import pallas as pl
from jax.experimental.pallas import tpu as pltpu

_N_E = 8192
_E_DIM = 32
_K = _N_E + 1
_KPAD = 9216
_W = 384
_B = 4
_T = 256
_BT = _B * _T
_KT = 1152
_EPSN = 1e-06 / _N_E
_BETA = 0.25


def _zsq_tree(zz):
    a = ((zz[:, 8:16] + zz[:, 0:8]) + zz[:, 16:24]) + zz[:, 24:32]
    b = a[:, 0:4] + a[:, 4:8]
    c = b[:, 0:2] + b[:, 2:4]
    return c[:, 0:1] + c[:, 1:2]


def _fused_kernel(zf_ref, emb_ref, embt_ref,
                  zq_ref, inds_ref, loss_ref, v_ref,
                  enc0_vmem, enc0_smem, sem, dwin_ref, oh_ref):
    zf = zf_ref[...]
    embt = embt_ref[...]
    zsq = _zsq_tree(zf * zf)

    z0 = jnp.concatenate(
        [zf[0:1], zf[_T:_T + 1], zf[2 * _T:2 * _T + 1], zf[3 * _T:3 * _T + 1],
         jnp.zeros((_T - _B, _E_DIM), jnp.float32)], axis=0)
    bsq = jnp.sum(embt * embt, axis=0, keepdims=True)
    zsq0 = jnp.concatenate(
        [zsq[0:1], zsq[_T:_T + 1], zsq[2 * _T:2 * _T + 1],
         zsq[3 * _T:3 * _T + 1]], axis=0)
    mm0 = jnp.dot(z0, embt, preferred_element_type=jnp.float32)
    d0 = (zsq0 + bsq) - 2.0 * mm0[:_B]
    kidx0 = jax.lax.broadcasted_iota(jnp.int32, (_B, _KPAD), 1)
    d0 = jnp.where(kidx0 < _K, d0, jnp.float32(1e30))
    rowmin = jnp.min(d0, axis=1, keepdims=True)
    idx = jnp.min(jnp.where(d0 == rowmin, kidx0, _KPAD), axis=1, keepdims=True)
    enc0 = jnp.clip(idx, 0, _N_E - 1)

    enc0_vmem[...] = enc0
    cp = pltpu.make_async_copy(enc0_vmem, enc0_smem, sem)
    cp.start()
    cp.wait()
    ws_list = [enc0_smem[b, 0] for b in range(_B)]

    ew_list = []
    for b in range(_B):
        ew = emb_ref[pl.ds(ws_list[b], _W), :]
        ew_list.append(ew)
        ewt = jnp.transpose(ew)
        wsq = jnp.sum(ewt * ewt, axis=0, keepdims=True)
        zb = zf[b * _T:(b + 1) * _T, :]
        mmb = jnp.dot(zb, ewt, preferred_element_type=jnp.float32)
        dwin = (zsq[b * _T:(b + 1) * _T, :] + wsq) - 2.0 * mmb
        dwin_ref[:, b, :] = dwin

    rowi = jax.lax.broadcasted_iota(jnp.int32, (_B, 1), 0)
    ws_vec = jnp.full((_B, 1), ws_list[0], dtype=jnp.int32)
    for b in range(1, _B):
        ws_vec = jnp.where(rowi == b, ws_list[b], ws_vec)
    rmax = jnp.int32(_N_E - 1) - ws_vec

    lane_w = jax.lax.broadcasted_iota(jnp.int32, (_B, _W), 1)
    adv_ok = lane_w < rmax
    h0 = (lane_w == 0).astype(jnp.float32)
    oh_ref[0, :, :] = h0

    def step(t, h):
        dt = dwin_ref[pl.ds(t, 1), :, :].reshape(_B, _W)
        dtn = pltpu.roll(dt, _W - 1, 1)
        a = jnp.logical_and(dt > dtn, adv_ok)
        adv_h = jnp.where(a, h, 0.0)
        h = (h - adv_h) + pltpu.roll(adv_h, 1, 1)
        oh_ref[pl.ds(t, 1), :, :] = h.reshape(1, _B, _W)
        return h

    jax.lax.fori_loop(1, _T, step, h0)

    ohall = oh_ref[...].reshape(_BT, _W)
    jcol = jax.lax.broadcasted_iota(jnp.int32, (_W, 1), 0).astype(jnp.float32)
    rvals = jnp.dot(ohall, jcol, preferred_element_type=jnp.float32)
    rint = rvals.astype(jnp.int32).reshape(_T, _B)
    inds = jnp.transpose(rint) + ws_vec
    inds_ref[...] = inds

    mx = jnp.max(inds)
    mn = jnp.min(inds)
    v_ref[0, 0] = mx - mn

    zq_parts = []
    for b in range(_B):
        oh_b = oh_ref[:, b, :].reshape(_T, _W)
        zq_parts.append(jnp.dot(oh_b, ew_list[b],
                                preferred_element_type=jnp.float32))
    zq = jnp.concatenate(zq_parts, axis=0)
    zq_ref[...] = zf + (zq - zf)

    diff = zf - zq
    dsel = jnp.sum(diff * diff, axis=1, keepdims=True)

    zf2 = zf + zf
    dselc = (dsel + jnp.float32(_EPSN)) - zsq
    acc = jnp.zeros((_BT, 1), dtype=jnp.float32)
    for i in range(_KPAD // _KT):
        et = embt[:, i * _KT:(i + 1) * _KT]
        bsq_t = bsq[:, i * _KT:(i + 1) * _KT]
        kidx = jax.lax.broadcasted_iota(jnp.int32, (1, _KT), 1) + i * _KT
        bsq_m = jnp.where(kidx < _K, bsq_t, jnp.float32(1e30))
        mm2 = jnp.dot(zf2, et, preferred_element_type=jnp.float32)
        term = jnp.maximum((dselc + mm2) - bsq_m, 0.0)
        acc = acc + jnp.sum(term, axis=1, keepdims=True)
    total = jnp.sum(acc)
    lmean = total / jnp.float32(_B * _T * _K)
    loss_ref[0, 0] = jnp.float32(_BETA) * lmean + lmean


@jax.jit
def kernel(z, embedding_weight):
    zf = z.reshape(_BT, _E_DIM)
    embp = jnp.pad(embedding_weight, ((0, _KPAD - _K), (0, 0)))
    embt = embp.T

    zq, inds, loss, v = pl.pallas_call(
        _fused_kernel,
        in_specs=[
            pl.BlockSpec(memory_space=pltpu.VMEM),
            pl.BlockSpec(memory_space=pltpu.VMEM),
            pl.BlockSpec(memory_space=pltpu.VMEM),
        ],
        out_shape=(
            jax.ShapeDtypeStruct((_BT, _E_DIM), jnp.float32),
            jax.ShapeDtypeStruct((_B, _T), jnp.int32),
            jax.ShapeDtypeStruct((1, 1), jnp.float32),
            jax.ShapeDtypeStruct((1, 1), jnp.int32),
        ),
        out_specs=(
            pl.BlockSpec(memory_space=pltpu.VMEM),
            pl.BlockSpec(memory_space=pltpu.VMEM),
            pl.BlockSpec(memory_space=pltpu.SMEM),
            pl.BlockSpec(memory_space=pltpu.SMEM),
        ),
        scratch_shapes=[
            pltpu.VMEM((_B, 1), jnp.int32),
            pltpu.SMEM((_B, 1), jnp.int32),
            pltpu.SemaphoreType.DMA,
            pltpu.VMEM((_T, _B, _W), jnp.float32),
            pltpu.VMEM((_T, _B, _W), jnp.float32),
        ],
    )(zf, embp, embt)

    return (zq.reshape(z.shape), loss.reshape(()), inds, v.reshape(()))


# speedup vs baseline: 1.3121x; 1.3121x over previous
import jax
import jax.numpy as jnp
from jax.experimental import pallas as pl
from jax.experimental.pallas import tpu as pltpu

_N_E = 8192
_E_DIM = 32
_K = _N_E + 1
_KPAD = 9216
_W = 384
_B = 4
_T = 256
_BT = _B * _T
_KT = 1152
_EPSN = 1e-06 / _N_E
_BETA = 0.25


def _zsq_tree(zz):
    a = ((zz[:, 8:16] + zz[:, 0:8]) + zz[:, 16:24]) + zz[:, 24:32]
    b = a[:, 0:4] + a[:, 4:8]
    c = b[:, 0:2] + b[:, 2:4]
    return c[:, 0:1] + c[:, 1:2]


def _fused_kernel(zf_ref, emb_ref, embt_ref,
                  zq_ref, inds_ref, loss_ref, v_ref,
                  enc0_vmem, enc0_smem, sem, dwin_ref, adv_ref):
    zf = zf_ref[...]
    embt = embt_ref[...]
    zsq = _zsq_tree(zf * zf)

    z0 = jnp.concatenate(
        [zf[0:1], zf[_T:_T + 1], zf[2 * _T:2 * _T + 1], zf[3 * _T:3 * _T + 1],
         jnp.zeros((_T - _B, _E_DIM), jnp.float32)], axis=0)
    bsq = jnp.sum(embt * embt, axis=0, keepdims=True)
    zsq0 = jnp.concatenate(
        [zsq[0:1], zsq[_T:_T + 1], zsq[2 * _T:2 * _T + 1],
         zsq[3 * _T:3 * _T + 1]], axis=0)
    mm0 = jnp.dot(z0, embt, preferred_element_type=jnp.float32)
    d0 = (zsq0 + bsq) - 2.0 * mm0[:_B]
    kidx0 = jax.lax.broadcasted_iota(jnp.int32, (_B, _KPAD), 1)
    d0 = jnp.where(kidx0 < _K, d0, jnp.float32(1e30))
    rowmin = jnp.min(d0, axis=1, keepdims=True)
    idx = jnp.min(jnp.where(d0 == rowmin, kidx0, _KPAD), axis=1, keepdims=True)
    enc0 = jnp.clip(idx, 0, _N_E - 1)

    enc0_vmem[...] = enc0
    cp = pltpu.make_async_copy(enc0_vmem, enc0_smem, sem)
    cp.start()
    cp.wait()
    ws_list = [enc0_smem[b, 0] for b in range(_B)]

    ew_list = []
    for b in range(_B):
        ew = emb_ref[pl.ds(ws_list[b], _W), :]
        ew_list.append(ew)
        ewt = jnp.transpose(ew)
        wsq = jnp.sum(ewt * ewt, axis=0, keepdims=True)
        zb = zf[b * _T:(b + 1) * _T, :]
        mmb = jnp.dot(zb, ewt, preferred_element_type=jnp.float32)
        dwin = (zsq[b * _T:(b + 1) * _T, :] + wsq) - 2.0 * mmb
        dwin_ref[:, b, :] = dwin

    rowi = jax.lax.broadcasted_iota(jnp.int32, (_B, 1), 0)
    ws_vec = jnp.full((_B, 1), ws_list[0], dtype=jnp.int32)
    for b in range(1, _B):
        ws_vec = jnp.where(rowi == b, ws_list[b], ws_vec)
    rmax = jnp.int32(_N_E - 1) - ws_vec

    lane_w = jax.lax.broadcasted_iota(jnp.int32, (_B, _W), 1)
    adv_ok = lane_w < rmax
    dall = dwin_ref[...]
    dnext = pltpu.roll(dall, _W - 1, 2)
    adv = jnp.logical_and(dall > dnext, adv_ok[None]).astype(jnp.int32)
    adv_ref[...] = adv

    lane_t = jax.lax.broadcasted_iota(jnp.int32, (_B, _T), 1)

    def step(t, carry):
        r, racc = carry
        at = adv_ref[pl.ds(t, 1), :, :].reshape(_B, _W)
        bit = jnp.sum(jnp.where(lane_w == r, at, 0), axis=1, keepdims=True)
        r = r + bit
        racc = jnp.where(lane_t == t, r, racc)
        return r, racc

    r0 = jnp.zeros((_B, 1), dtype=jnp.int32)
    racc0 = jnp.zeros((_B, _T), dtype=jnp.int32)
    _, racc = jax.lax.fori_loop(1, _T, step, (r0, racc0))

    inds = racc + ws_vec
    inds_ref[...] = inds

    mx = jnp.max(inds)
    mn = jnp.min(inds)
    v_ref[0, 0] = mx - mn

    zq_parts = []
    for b in range(_B):
        rcol = racc[b].reshape(_T, 1)
        oh_b = (jax.lax.broadcasted_iota(jnp.int32, (_T, _W), 1)
                == rcol).astype(jnp.float32)
        zq_parts.append(jnp.dot(oh_b, ew_list[b],
                                preferred_element_type=jnp.float32))
    zq = jnp.concatenate(zq_parts, axis=0)
    zq_ref[...] = zf + (zq - zf)

    diff = zf - zq
    dsel = jnp.sum(diff * diff, axis=1, keepdims=True)

    zf2 = zf + zf
    dselc = (dsel + jnp.float32(_EPSN)) - zsq
    acc = jnp.zeros((_BT, 1), dtype=jnp.float32)
    for i in range(_KPAD // _KT):
        et = embt[:, i * _KT:(i + 1) * _KT]
        bsq_t = bsq[:, i * _KT:(i + 1) * _KT]
        kidx = jax.lax.broadcasted_iota(jnp.int32, (1, _KT), 1) + i * _KT
        bsq_m = jnp.where(kidx < _K, bsq_t, jnp.float32(1e30))
        mm2 = jnp.dot(zf2, et, preferred_element_type=jnp.float32)
        term = jnp.maximum((dselc + mm2) - bsq_m, 0.0)
        acc = acc + jnp.sum(term, axis=1, keepdims=True)
    total = jnp.sum(acc)
    lmean = total / jnp.float32(_B * _T * _K)
    loss_ref[0, 0] = jnp.float32(_BETA) * lmean + lmean


@jax.jit
def kernel(z, embedding_weight):
    zf = z.reshape(_BT, _E_DIM)
    embp = jnp.pad(embedding_weight, ((0, _KPAD - _K), (0, 0)))
    embt = embp.T

    zq, inds, loss, v = pl.pallas_call(
        _fused_kernel,
        in_specs=[
            pl.BlockSpec(memory_space=pltpu.VMEM),
            pl.BlockSpec(memory_space=pltpu.VMEM),
            pl.BlockSpec(memory_space=pltpu.VMEM),
        ],
        out_shape=(
            jax.ShapeDtypeStruct((_BT, _E_DIM), jnp.float32),
            jax.ShapeDtypeStruct((_B, _T), jnp.int32),
            jax.ShapeDtypeStruct((1, 1), jnp.float32),
            jax.ShapeDtypeStruct((1, 1), jnp.int32),
        ),
        out_specs=(
            pl.BlockSpec(memory_space=pltpu.VMEM),
            pl.BlockSpec(memory_space=pltpu.VMEM),
            pl.BlockSpec(memory_space=pltpu.SMEM),
            pl.BlockSpec(memory_space=pltpu.SMEM),
        ),
        scratch_shapes=[
            pltpu.VMEM((_B, 1), jnp.int32),
            pltpu.SMEM((_B, 1), jnp.int32),
            pltpu.SemaphoreType.DMA,
            pltpu.VMEM((_T, _B, _W), jnp.float32),
            pltpu.VMEM((_T, _B, _W), jnp.int32),
        ],
    )(zf, embp, embt)

    return (zq.reshape(z.shape), loss.reshape(()), inds, v.reshape(()))


# speedup vs baseline: 1.3444x; 1.0246x over previous
import jax
import jax.numpy as jnp
from jax.experimental import pallas as pl
from jax.experimental.pallas import tpu as pltpu

_N_E = 8192
_E_DIM = 32
_K = _N_E + 1
_KPAD = 9216
_W = 384
_B = 4
_T = 256
_BT = _B * _T
_KT = 1152
_EPSN = 1e-06 / _N_E
_BETA = 0.25


def _zsq_tree(zz):
    a = ((zz[:, 8:16] + zz[:, 0:8]) + zz[:, 16:24]) + zz[:, 24:32]
    b = a[:, 0:4] + a[:, 4:8]
    c = b[:, 0:2] + b[:, 2:4]
    return c[:, 0:1] + c[:, 1:2]


def _fused_kernel(zf_ref, emb_ref, embt_ref,
                  zq_ref, inds_ref, loss_ref, v_ref,
                  enc0_vmem, enc0_smem, sem, dwin_ref, adv_ref):
    zf = zf_ref[...]
    embt = embt_ref[...]
    zsq = _zsq_tree(zf * zf)

    z0 = jnp.concatenate(
        [zf[0:1], zf[_T:_T + 1], zf[2 * _T:2 * _T + 1], zf[3 * _T:3 * _T + 1],
         jnp.zeros((_T - _B, _E_DIM), jnp.float32)], axis=0)
    bsq = jnp.sum(embt * embt, axis=0, keepdims=True)
    zsq0 = jnp.concatenate(
        [zsq[0:1], zsq[_T:_T + 1], zsq[2 * _T:2 * _T + 1],
         zsq[3 * _T:3 * _T + 1]], axis=0)
    mm0 = jnp.dot(z0, embt, preferred_element_type=jnp.float32)
    d0 = (zsq0 + bsq) - 2.0 * mm0[:_B]
    kidx0 = jax.lax.broadcasted_iota(jnp.int32, (_B, _KPAD), 1)
    d0 = jnp.where(kidx0 < _K, d0, jnp.float32(1e30))
    rowmin = jnp.min(d0, axis=1, keepdims=True)
    idx = jnp.min(jnp.where(d0 == rowmin, kidx0, _KPAD), axis=1, keepdims=True)
    enc0 = jnp.clip(idx, 0, _N_E - 1)

    enc0_vmem[...] = enc0
    cp = pltpu.make_async_copy(enc0_vmem, enc0_smem, sem)
    cp.start()
    cp.wait()
    ws_list = [enc0_smem[b, 0] for b in range(_B)]

    ew_list = []
    for b in range(_B):
        ew = emb_ref[pl.ds(ws_list[b], _W), :]
        ew_list.append(ew)
        ewt = jnp.transpose(ew)
        wsq = jnp.sum(ewt * ewt, axis=0, keepdims=True)
        zb = zf[b * _T:(b + 1) * _T, :]
        mmb = jnp.dot(zb, ewt, preferred_element_type=jnp.float32)
        dwin = (zsq[b * _T:(b + 1) * _T, :] + wsq) - 2.0 * mmb
        dwin_ref[:, b, :] = dwin

    rowi = jax.lax.broadcasted_iota(jnp.int32, (_B, 1), 0)
    ws_vec = jnp.full((_B, 1), ws_list[0], dtype=jnp.int32)
    for b in range(1, _B):
        ws_vec = jnp.where(rowi == b, ws_list[b], ws_vec)
    rmax = jnp.int32(_N_E - 1) - ws_vec

    lane_w = jax.lax.broadcasted_iota(jnp.int32, (_B, _W), 1)
    adv_ok = lane_w < rmax
    dall = dwin_ref[...]
    dnext = pltpu.roll(dall, _W - 1, 2)
    adv = jnp.logical_and(dall > dnext, adv_ok[None]).astype(jnp.int32)
    adv_ref[...] = adv

    r = jnp.zeros((_B, 1), dtype=jnp.int32)
    r_cols = [r]
    for t in range(1, _T):
        at = adv_ref[t, :, :]
        bit = jnp.sum(jnp.where(lane_w == r, at, 0), axis=1, keepdims=True)
        r = r + bit
        r_cols.append(r)
    racc = jnp.concatenate(r_cols, axis=1)

    inds = racc + ws_vec
    inds_ref[...] = inds

    mx = jnp.max(inds)
    mn = jnp.min(inds)
    v_ref[0, 0] = mx - mn

    zq_parts = []
    for b in range(_B):
        rcol = racc[b].reshape(_T, 1)
        oh_b = (jax.lax.broadcasted_iota(jnp.int32, (_T, _W), 1)
                == rcol).astype(jnp.float32)
        zq_parts.append(jnp.dot(oh_b, ew_list[b],
                                preferred_element_type=jnp.float32))
    zq = jnp.concatenate(zq_parts, axis=0)
    zq_ref[...] = zf + (zq - zf)

    diff = zf - zq
    dsel = jnp.sum(diff * diff, axis=1, keepdims=True)

    zf2 = zf + zf
    dselc = (dsel + jnp.float32(_EPSN)) - zsq
    acc = jnp.zeros((_BT, 1), dtype=jnp.float32)
    for i in range(_KPAD // _KT):
        et = embt[:, i * _KT:(i + 1) * _KT]
        bsq_t = bsq[:, i * _KT:(i + 1) * _KT]
        kidx = jax.lax.broadcasted_iota(jnp.int32, (1, _KT), 1) + i * _KT
        bsq_m = jnp.where(kidx < _K, bsq_t, jnp.float32(1e30))
        mm2 = jnp.dot(zf2, et, preferred_element_type=jnp.float32)
        term = jnp.maximum((dselc + mm2) - bsq_m, 0.0)
        acc = acc + jnp.sum(term, axis=1, keepdims=True)
    total = jnp.sum(acc)
    lmean = total / jnp.float32(_B * _T * _K)
    loss_ref[0, 0] = jnp.float32(_BETA) * lmean + lmean


@jax.jit
def kernel(z, embedding_weight):
    zf = z.reshape(_BT, _E_DIM)
    embp = jnp.pad(embedding_weight, ((0, _KPAD - _K), (0, 0)))
    embt = embp.T

    zq, inds, loss, v = pl.pallas_call(
        _fused_kernel,
        in_specs=[
            pl.BlockSpec(memory_space=pltpu.VMEM),
            pl.BlockSpec(memory_space=pltpu.VMEM),
            pl.BlockSpec(memory_space=pltpu.VMEM),
        ],
        out_shape=(
            jax.ShapeDtypeStruct((_BT, _E_DIM), jnp.float32),
            jax.ShapeDtypeStruct((_B, _T), jnp.int32),
            jax.ShapeDtypeStruct((1, 1), jnp.float32),
            jax.ShapeDtypeStruct((1, 1), jnp.int32),
        ),
        out_specs=(
            pl.BlockSpec(memory_space=pltpu.VMEM),
            pl.BlockSpec(memory_space=pltpu.VMEM),
            pl.BlockSpec(memory_space=pltpu.SMEM),
            pl.BlockSpec(memory_space=pltpu.SMEM),
        ),
        scratch_shapes=[
            pltpu.VMEM((_B, 1), jnp.int32),
            pltpu.SMEM((_B, 1), jnp.int32),
            pltpu.SemaphoreType.DMA,
            pltpu.VMEM((_T, _B, _W), jnp.float32),
            pltpu.VMEM((_T, _B, _W), jnp.int32),
        ],
    )(zf, embp, embt)

    return (zq.reshape(z.shape), loss.reshape(()), inds, v.reshape(()))


# speedup vs baseline: 1.8385x; 1.3675x over previous
import jax
import jax.numpy as jnp
from jax.experimental import pallas as pl
from jax.experimental.pallas import tpu as pltpu

_N_E = 8192
_E_DIM = 32
_K = _N_E + 1
_KPAD = 9216
_W = 384
_B = 4
_T = 256
_BT = _B * _T
_KT = 1152
_SUB = 16
_NB = 16
_EPSN = 1e-06 / _N_E
_BETA = 0.25


def _zsq_tree(zz):
    a = ((zz[:, 8:16] + zz[:, 0:8]) + zz[:, 16:24]) + zz[:, 24:32]
    b = a[:, 0:4] + a[:, 4:8]
    c = b[:, 0:2] + b[:, 2:4]
    return c[:, 0:1] + c[:, 1:2]


def _fused_kernel(zf_ref, emb_ref, embt_ref,
                  zq_ref, inds_ref, loss_ref, v_ref,
                  enc0_vmem, enc0_smem, sem, dwin_ref, adv_ref, racc_ref):
    zf = zf_ref[...]
    embt = embt_ref[...]
    zsq = _zsq_tree(zf * zf)

    z0 = jnp.concatenate(
        [zf[0:1], zf[_T:_T + 1], zf[2 * _T:2 * _T + 1], zf[3 * _T:3 * _T + 1],
         jnp.zeros((_T - _B, _E_DIM), jnp.float32)], axis=0)
    bsq = jnp.sum(embt * embt, axis=0, keepdims=True)
    zsq0 = jnp.concatenate(
        [zsq[0:1], zsq[_T:_T + 1], zsq[2 * _T:2 * _T + 1],
         zsq[3 * _T:3 * _T + 1]], axis=0)
    mm0 = jnp.dot(z0, embt, preferred_element_type=jnp.float32)
    d0 = (zsq0 + bsq) - 2.0 * mm0[:_B]
    kidx0 = jax.lax.broadcasted_iota(jnp.int32, (_B, _KPAD), 1)
    d0 = jnp.where(kidx0 < _K, d0, jnp.float32(1e30))
    rowmin = jnp.min(d0, axis=1, keepdims=True)
    idx = jnp.min(jnp.where(d0 == rowmin, kidx0, _KPAD), axis=1, keepdims=True)
    enc0 = jnp.clip(idx, 0, _N_E - 1)

    enc0_vmem[...] = enc0
    cp = pltpu.make_async_copy(enc0_vmem, enc0_smem, sem)
    cp.start()
    cp.wait()
    ws_list = [enc0_smem[b, 0] for b in range(_B)]

    ew_list = []
    for b in range(_B):
        ew = emb_ref[pl.ds(ws_list[b], _W), :]
        ew_list.append(ew)
        ewt = jnp.transpose(ew)
        wsq = jnp.sum(ewt * ewt, axis=0, keepdims=True)
        zb = zf[b * _T:(b + 1) * _T, :]
        mmb = jnp.dot(zb, ewt, preferred_element_type=jnp.float32)
        dwin = (zsq[b * _T:(b + 1) * _T, :] + wsq) - 2.0 * mmb
        dwin_ref[:, b, :] = dwin

    rowi = jax.lax.broadcasted_iota(jnp.int32, (_B, 1), 0)
    ws_vec = jnp.full((_B, 1), ws_list[0], dtype=jnp.int32)
    for b in range(1, _B):
        ws_vec = jnp.where(rowi == b, ws_list[b], ws_vec)
    rmax = jnp.int32(_N_E - 1) - ws_vec

    lane_w = jax.lax.broadcasted_iota(jnp.int32, (_B, _W), 1)
    adv_ok = lane_w < rmax
    dall = dwin_ref[...]
    dnext = pltpu.roll(dall, _W - 1, 2)
    adv = jnp.logical_and(dall > dnext, adv_ok[None]).astype(jnp.int32)
    t_iota = jax.lax.broadcasted_iota(jnp.int32, (_T, _B, _W), 0)
    adv = jnp.where(t_iota == 0, 0, adv)
    advb = adv.reshape(_NB, _SUB, _B, _W)
    for g in range(_NB):
        adv_ref[:, g, :, :] = advb[g]

    dsp = adv_ref[0, :, :, :]
    for s in range(1, _SUB):
        a_s = adv_ref[s, :, :, :]
        bit = jnp.where(dsp == 0, a_s, 0)
        for delta in range(1, s + 1):
            sh = pltpu.roll(a_s, _W - delta, 2)
            bit = bit + jnp.where(dsp == delta, sh, 0)
        dsp = dsp + bit

    lane_w3 = jax.lax.broadcasted_iota(jnp.int32, (_NB, _B, _W), 2)
    r = jnp.zeros((_B, 1), dtype=jnp.int32)
    bstart_rows = []
    for g in range(_NB):
        bstart_rows.append(jnp.transpose(r))
        jump = jnp.sum(jnp.where(lane_w == r, dsp[g], 0),
                       axis=1, keepdims=True)
        r = r + jump
    bcur = jnp.concatenate(bstart_rows, axis=0)

    for s in range(_SUB):
        a_s = adv_ref[s, :, :, :]
        bit = jnp.sum(jnp.where(lane_w3 == bcur[:, :, None], a_s, 0),
                      axis=2)
        bcur = bcur + bit
        racc_ref[:, :, s] = bcur
    racc = jnp.transpose(racc_ref[...], (1, 0, 2)).reshape(_B, _T)

    inds = racc + ws_vec
    inds = racc + ws_vec
    inds_ref[...] = inds

    mx = jnp.max(inds)
    mn = jnp.min(inds)
    v_ref[0, 0] = mx - mn

    zq_parts = []
    for b in range(_B):
        rcol = racc[b].reshape(_T, 1)
        oh_b = (jax.lax.broadcasted_iota(jnp.int32, (_T, _W), 1)
                == rcol).astype(jnp.float32)
        zq_parts.append(jnp.dot(oh_b, ew_list[b],
                                preferred_element_type=jnp.float32))
    zq = jnp.concatenate(zq_parts, axis=0)
    zq_ref[...] = zf + (zq - zf)

    diff = zf - zq
    dsel = jnp.sum(diff * diff, axis=1, keepdims=True)

    zf2 = zf + zf
    dselc = (dsel + jnp.float32(_EPSN)) - zsq
    acc = jnp.zeros((_BT, 1), dtype=jnp.float32)
    for i in range(_KPAD // _KT):
        et = embt[:, i * _KT:(i + 1) * _KT]
        bsq_t = bsq[:, i * _KT:(i + 1) * _KT]
        kidx = jax.lax.broadcasted_iota(jnp.int32, (1, _KT), 1) + i * _KT
        bsq_m = jnp.where(kidx < _K, bsq_t, jnp.float32(1e30))
        mm2 = jnp.dot(zf2, et, preferred_element_type=jnp.float32)
        term = jnp.maximum((dselc + mm2) - bsq_m, 0.0)
        acc = acc + jnp.sum(term, axis=1, keepdims=True)
    total = jnp.sum(acc)
    lmean = total / jnp.float32(_B * _T * _K)
    loss_ref[0, 0] = jnp.float32(_BETA) * lmean + lmean


@jax.jit
def kernel(z, embedding_weight):
    zf = z.reshape(_BT, _E_DIM)
    embp = jnp.pad(embedding_weight, ((0, _KPAD - _K), (0, 0)))
    embt = embp.T

    zq, inds, loss, v = pl.pallas_call(
        _fused_kernel,
        in_specs=[
            pl.BlockSpec(memory_space=pltpu.VMEM),
            pl.BlockSpec(memory_space=pltpu.VMEM),
            pl.BlockSpec(memory_space=pltpu.VMEM),
        ],
        out_shape=(
            jax.ShapeDtypeStruct((_BT, _E_DIM), jnp.float32),
            jax.ShapeDtypeStruct((_B, _T), jnp.int32),
            jax.ShapeDtypeStruct((1, 1), jnp.float32),
            jax.ShapeDtypeStruct((1, 1), jnp.int32),
        ),
        out_specs=(
            pl.BlockSpec(memory_space=pltpu.VMEM),
            pl.BlockSpec(memory_space=pltpu.VMEM),
            pl.BlockSpec(memory_space=pltpu.SMEM),
            pl.BlockSpec(memory_space=pltpu.SMEM),
        ),
        scratch_shapes=[
            pltpu.VMEM((_B, 1), jnp.int32),
            pltpu.SMEM((_B, 1), jnp.int32),
            pltpu.SemaphoreType.DMA,
            pltpu.VMEM((_T, _B, _W), jnp.float32),
            pltpu.VMEM((_SUB, _NB, _B, _W), jnp.int32),
            pltpu.VMEM((_NB, _B, _SUB), jnp.int32),
        ],
    )(zf, embp, embt)

    return (zq.reshape(z.shape), loss.reshape(()), inds, v.reshape(()))
